# baseline (speedup 1.0000x reference)
import functools

import jax
import jax.numpy as jnp
from jax import lax
from jax.experimental import pallas as pl
from jax.experimental.pallas import tpu as pltpu
from jax.experimental.pallas import tpu_sc as plsc

_B, _N, _M, _K = 8, 20000, 128, 2
_NC, _NS, _L = 2, 16, 16
_NW = _NC * _NS
_CHUNK = (_B * _N) // _NW
_ITERS = (_CHUNK + _L - 1) // _L
_LAST_OFF = _CHUNK - _L
_WPB = _NW // _B
_LN2 = 0.6931471805599453
_SQRT2 = 1.4142135623730951


def _ln(x):
  bits = lax.bitcast_convert_type(x, jnp.int32)
  e = lax.shift_right_arithmetic(bits, 23) - 127
  mbits = lax.bitwise_or(lax.bitwise_and(bits, 0x007FFFFF), 0x3F800000)
  m = lax.bitcast_convert_type(mbits, jnp.float32)
  big = m > _SQRT2
  m = jnp.where(big, m * 0.5, m)
  ef = (e + jnp.where(big, 1, 0)).astype(jnp.float32)
  t = (m - 1.0) / (m + 1.0)
  t2 = t * t
  p = 1.0 / 7.0
  p = 0.2 + t2 * p
  p = 1.0 / 3.0 + t2 * p
  lnm = (2.0 * t) * (1.0 + t2 * p)
  return ef * _LN2 + lnm


@functools.cache
def _build_sc_encode():
  mesh = plsc.VectorSubcoreMesh(core_axis_name="c", subcore_axis_name="s")

  flat = jax.ShapeDtypeStruct((_B * _N,), jnp.float32)

  @functools.partial(
      pl.kernel,
      mesh=mesh,
      compiler_params=pltpu.CompilerParams(
          needs_layout_passes=False, use_tc_tiling_on_sc=False),
      out_type=[flat] * 8,
      scratch_types=[
          pltpu.VMEM((_M, 8), jnp.float32),
          pltpu.VMEM((_CHUNK,), jnp.int32),
          pltpu.VMEM((_CHUNK,), jnp.int32),
      ] + [pltpu.VMEM((_CHUNK,), jnp.float32)] * 12 + [
          pltpu.SemaphoreType.DMA,
          pltpu.SemaphoreType.DMA,
      ],
  )
  def _sc_encode(x1_hbm, y1_hbm, x2_hbm, y2_hbm, tab_hbm, ids_hbm, flg_hbm,
                 cls_hbm, clsm_hbm, hd0_hbm, ht0_hbm, hd1_hbm, ht1_hbm,
                 rm0_hbm, rm1_hbm,
                 tab_v, ids_v, flg_v,
                 x1_v, y1_v, x2_v, y2_v,
                 cls_v, clsm_v, hd0_v, ht0_v, hd1_v, ht1_v, rm0_v, rm1_v,
                 in_sem, out_sem):
    wid = lax.axis_index("s") * _NC + lax.axis_index("c")
    base = wid * _CHUNK
    b = wid // _WPB

    in_sl = pl.ds(base, _CHUNK)
    copies = [
        pltpu.async_copy(tab_hbm.at[pl.ds(b * _M, _M)], tab_v, in_sem),
        pltpu.async_copy(ids_hbm.at[in_sl], ids_v, in_sem),
        pltpu.async_copy(flg_hbm.at[in_sl], flg_v, in_sem),
        pltpu.async_copy(x1_hbm.at[in_sl], x1_v, in_sem),
        pltpu.async_copy(y1_hbm.at[in_sl], y1_v, in_sem),
        pltpu.async_copy(x2_hbm.at[in_sl], x2_v, in_sem),
        pltpu.async_copy(y2_hbm.at[in_sl], y2_v, in_sem),
    ]
    for c in copies:
      c.wait()

    def col(c):
      return jnp.full((_L,), c, jnp.int32)

    @plsc.parallel_loop(0, _ITERS, 1, unroll=8)
    def body(i):
      off = jnp.minimum(i * _L, _LAST_OFF)
      sl = pl.ds(off, _L)
      idv = ids_v[sl]
      flg = flg_v[sl]

      gcls = plsc.load_gather(tab_v, [idv, col(0)])
      fx0 = plsc.load_gather(tab_v, [idv, col(1)])
      fy0 = plsc.load_gather(tab_v, [idv, col(2)])
      fc0 = plsc.load_gather(tab_v, [idv, col(3)])
      fx1 = plsc.load_gather(tab_v, [idv, col(4)])
      fy1 = plsc.load_gather(tab_v, [idv, col(5)])
      fc1 = plsc.load_gather(tab_v, [idv, col(6)])
      x1 = x1_v[sl]
      y1 = y1_v[sl]
      x2 = x2_v[sl]
      y2 = y2_v[sl]

      pos = flg > 0
      ok = jnp.logical_and(pos, gcls != 0.0)
      f0p = fc0 > 0.0
      f1p = fc1 > 0.0
      cls = jnp.where(
          jnp.logical_and(ok, jnp.logical_and(f0p, f1p)), 1.0,
          jnp.where(ok, 0.0, -1.0))
      clsm = jnp.where(ok, 1.0, 0.0)

      cx = (x1 + x2) * 0.5
      w = x2 - x1
      h = y2 - y1
      inv_w = 1.0 / w
      inv_h = 1.0 / h

      ht0 = fy0 - y1
      hm0 = ht0 > 0.0
      htgt0 = jnp.where(hm0, _ln(jnp.maximum(ht0 * inv_h, 1e-30)), 0.0)
      hd0 = jnp.where(hm0, (fx0 - cx) * inv_w, 0.0)
      rm0 = jnp.where(
          jnp.logical_and(jnp.logical_and(pos, f0p), hm0), 1.0, 0.0)

      ht1 = fy1 - y1
      hm1 = ht1 > 0.0
      htgt1 = jnp.where(hm1, _ln(jnp.maximum(ht1 * inv_h, 1e-30)), 0.0)
      hd1 = jnp.where(hm1, (fx1 - cx) * inv_w, 0.0)
      rm1 = jnp.where(
          jnp.logical_and(jnp.logical_and(pos, f1p), hm1), 1.0, 0.0)

      cls_v[sl] = cls
      clsm_v[sl] = clsm
      hd0_v[sl] = hd0
      ht0_v[sl] = htgt0
      hd1_v[sl] = hd1
      ht1_v[sl] = htgt1
      rm0_v[sl] = rm0
      rm1_v[sl] = rm1

    out_sl = pl.ds(base, _CHUNK)
    out_copies = [
        pltpu.async_copy(cls_v, cls_hbm.at[out_sl], out_sem),
        pltpu.async_copy(clsm_v, clsm_hbm.at[out_sl], out_sem),
        pltpu.async_copy(hd0_v, hd0_hbm.at[out_sl], out_sem),
        pltpu.async_copy(ht0_v, ht0_hbm.at[out_sl], out_sem),
        pltpu.async_copy(hd1_v, hd1_hbm.at[out_sl], out_sem),
        pltpu.async_copy(ht1_v, ht1_hbm.at[out_sl], out_sem),
        pltpu.async_copy(rm0_v, rm0_hbm.at[out_sl], out_sem),
        pltpu.async_copy(rm1_v, rm1_hbm.at[out_sl], out_sem),
    ]
    for c in out_copies:
      c.wait()

  return _sc_encode


def kernel(boxes, gt_boxes, gt_flanks, match_pos_flag, match_gt_id):
  B, N, _ = boxes.shape
  M = gt_boxes.shape[1]
  tab = jnp.concatenate(
      [gt_boxes[..., 4:5],
       gt_flanks[:, :, 0, :],
       gt_flanks[:, :, 1, :],
       jnp.zeros((B, M, 1), jnp.float32)], axis=-1)
  cls, clsm, hd0, ht0, hd1, ht1, rm0, rm1 = _build_sc_encode()(
      boxes[..., 0].reshape(B * N),
      boxes[..., 1].reshape(B * N),
      boxes[..., 2].reshape(B * N),
      boxes[..., 3].reshape(B * N),
      tab.reshape(B * M, 8),
      match_gt_id.astype(jnp.int32).reshape(B * N),
      match_pos_flag.astype(jnp.int32).reshape(B * N),
  )
  cls_label = cls.reshape(B, N)
  cls_label_mask = clsm.reshape(B, N)
  reg_label = jnp.stack(
      [jnp.stack([hd0.reshape(B, N), ht0.reshape(B, N)], axis=-1),
       jnp.stack([hd1.reshape(B, N), ht1.reshape(B, N)], axis=-1)], axis=2)
  rm = jnp.stack(
      [rm0.reshape(B, N) > 0.0, rm1.reshape(B, N) > 0.0], axis=2)[..., None]
  reg_label_mask = jnp.concatenate([rm, rm], axis=-1)
  return (cls_label, cls_label_mask, reg_label, reg_label_mask)


# speedup vs baseline: 1.3431x; 1.3431x over previous
import functools

import jax
import jax.numpy as jnp
from jax import lax
from jax.experimental import pallas as pl
from jax.experimental.pallas import tpu as pltpu
from jax.experimental.pallas import tpu_sc as plsc

_B, _N, _M, _K = 8, 20000, 128, 2
_NC, _NS, _L = 2, 16, 16
_NW = _NC * _NS
_CHUNK = (_B * _N) // _NW
_ITERS = (_CHUNK + _L - 1) // _L
_LAST_OFF = _CHUNK - _L
_WPB = _NW // _B
_LN2 = 0.6931471805599453
_SQRT2 = 1.4142135623730951


def _ln(x):
  bits = lax.bitcast_convert_type(x, jnp.int32)
  e = lax.shift_right_arithmetic(bits, 23) - 127
  mbits = lax.bitwise_or(lax.bitwise_and(bits, 0x007FFFFF), 0x3F800000)
  m = lax.bitcast_convert_type(mbits, jnp.float32)
  big = m > _SQRT2
  m = jnp.where(big, m * 0.5, m)
  ef = (e + jnp.where(big, 1, 0)).astype(jnp.float32)
  t = (m - 1.0) / (m + 1.0)
  t2 = t * t
  p = 1.0 / 7.0
  p = 0.2 + t2 * p
  p = 1.0 / 3.0 + t2 * p
  lnm = (2.0 * t) * (1.0 + t2 * p)
  return ef * _LN2 + lnm


@functools.cache
def _build_sc_encode():
  mesh = plsc.VectorSubcoreMesh(core_axis_name="c", subcore_axis_name="s")

  flat = jax.ShapeDtypeStruct((_B * _N,), jnp.float32)

  @functools.partial(
      pl.kernel,
      mesh=mesh,
      compiler_params=pltpu.CompilerParams(
          needs_layout_passes=False, use_tc_tiling_on_sc=False),
      out_type=[flat] * 8,
      scratch_types=[
          pltpu.VMEM((_M, 8), jnp.float32),
          pltpu.VMEM((_CHUNK,), jnp.int32),
          pltpu.VMEM((_CHUNK,), jnp.int32),
      ] + [pltpu.VMEM((_CHUNK,), jnp.float32)] * 12 + [
          pltpu.SemaphoreType.DMA,
          pltpu.SemaphoreType.DMA,
      ],
  )
  def _sc_encode(x1_hbm, y1_hbm, x2_hbm, y2_hbm, tab_hbm, ids_hbm, flg_hbm,
                 cls_hbm, clsm_hbm, hd0_hbm, ht0_hbm, hd1_hbm, ht1_hbm,
                 rm0_hbm, rm1_hbm,
                 tab_v, ids_v, flg_v,
                 x1_v, y1_v, x2_v, y2_v,
                 cls_v, clsm_v, hd0_v, ht0_v, hd1_v, ht1_v, rm0_v, rm1_v,
                 in_sem, out_sem):
    wid = lax.axis_index("s") * _NC + lax.axis_index("c")
    base = wid * _CHUNK
    b = wid // _WPB

    in_sl = pl.ds(base, _CHUNK)
    copies = [
        pltpu.async_copy(tab_hbm.at[pl.ds(b * _M, _M)], tab_v, in_sem),
        pltpu.async_copy(ids_hbm.at[in_sl], ids_v, in_sem),
        pltpu.async_copy(flg_hbm.at[in_sl], flg_v, in_sem),
        pltpu.async_copy(x1_hbm.at[in_sl], x1_v, in_sem),
        pltpu.async_copy(y1_hbm.at[in_sl], y1_v, in_sem),
        pltpu.async_copy(x2_hbm.at[in_sl], x2_v, in_sem),
        pltpu.async_copy(y2_hbm.at[in_sl], y2_v, in_sem),
    ]
    for c in copies:
      c.wait()

    def col(c):
      return jnp.full((_L,), c, jnp.int32)

    @plsc.parallel_loop(0, _ITERS, 1, unroll=8)
    def body(i):
      off = jnp.minimum(i * _L, _LAST_OFF)
      sl = pl.ds(off, _L)
      idv = ids_v[sl]
      flg = flg_v[sl]

      gcls = plsc.load_gather(tab_v, [idv, col(0)])
      fx0 = plsc.load_gather(tab_v, [idv, col(1)])
      fy0 = plsc.load_gather(tab_v, [idv, col(2)])
      fc0 = plsc.load_gather(tab_v, [idv, col(3)])
      fx1 = plsc.load_gather(tab_v, [idv, col(4)])
      fy1 = plsc.load_gather(tab_v, [idv, col(5)])
      fc1 = plsc.load_gather(tab_v, [idv, col(6)])
      x1 = x1_v[sl]
      y1 = y1_v[sl]
      x2 = x2_v[sl]
      y2 = y2_v[sl]

      pos = flg > 0
      force = jnp.logical_or(jnp.logical_not(pos), gcls == 0.0)
      fc0p = jnp.where(force, -1.0, fc0)
      fc1p = jnp.where(force, -1.0, fc1)
      pos_mask = jnp.logical_and(fc0p > 0.0, fc1p > 0.0)
      neg_mask = jnp.logical_or(fc0p == 0.0, fc1p == 0.0)
      ign_mask = jnp.logical_or(fc0p < 0.0, fc1p < 0.0)
      cls = jnp.where(pos_mask, 1.0, 0.0)
      cls = jnp.where(neg_mask, 0.0, cls)
      cls = jnp.where(ign_mask, -1.0, cls)
      clsm = jnp.where(cls >= 0.0, 1.0, 0.0)

      cx = (x1 + x2) * 0.5
      w = x2 - x1
      h = y2 - y1
      bm = jnp.logical_and(w > 0.0, h > 0.0)
      inv_w = 1.0 / w
      inv_h = 1.0 / h

      ht0 = fy0 - y1
      hm0 = jnp.logical_and(bm, ht0 > 0.0)
      htgt0 = jnp.where(hm0, _ln(jnp.maximum(ht0 * inv_h, 1e-30)), 0.0)
      hd0 = jnp.where(hm0, (fx0 - cx) * inv_w, 0.0)
      rm0 = jnp.where(
          jnp.logical_and(jnp.logical_and(pos, fc0 > 0.0), hm0), 1.0, 0.0)

      ht1 = fy1 - y1
      hm1 = jnp.logical_and(bm, ht1 > 0.0)
      htgt1 = jnp.where(hm1, _ln(jnp.maximum(ht1 * inv_h, 1e-30)), 0.0)
      hd1 = jnp.where(hm1, (fx1 - cx) * inv_w, 0.0)
      rm1 = jnp.where(
          jnp.logical_and(jnp.logical_and(pos, fc1 > 0.0), hm1), 1.0, 0.0)

      cls_v[sl] = cls
      clsm_v[sl] = clsm
      hd0_v[sl] = hd0
      ht0_v[sl] = htgt0
      hd1_v[sl] = hd1
      ht1_v[sl] = htgt1
      rm0_v[sl] = rm0
      rm1_v[sl] = rm1

    out_sl = pl.ds(base, _CHUNK)
    out_copies = [
        pltpu.async_copy(cls_v, cls_hbm.at[out_sl], out_sem),
        pltpu.async_copy(clsm_v, clsm_hbm.at[out_sl], out_sem),
        pltpu.async_copy(hd0_v, hd0_hbm.at[out_sl], out_sem),
        pltpu.async_copy(ht0_v, ht0_hbm.at[out_sl], out_sem),
        pltpu.async_copy(hd1_v, hd1_hbm.at[out_sl], out_sem),
        pltpu.async_copy(ht1_v, ht1_hbm.at[out_sl], out_sem),
        pltpu.async_copy(rm0_v, rm0_hbm.at[out_sl], out_sem),
        pltpu.async_copy(rm1_v, rm1_hbm.at[out_sl], out_sem),
    ]
    for c in out_copies:
      c.wait()

  return _sc_encode


def kernel(boxes, gt_boxes, gt_flanks, match_pos_flag, match_gt_id):
  B, N, _ = boxes.shape
  M = gt_boxes.shape[1]
  tab = jnp.concatenate(
      [gt_boxes[..., 4:5],
       gt_flanks[:, :, 0, :],
       gt_flanks[:, :, 1, :],
       jnp.zeros((B, M, 1), jnp.float32)], axis=-1)
  cls, clsm, hd0, ht0, hd1, ht1, rm0, rm1 = _build_sc_encode()(
      boxes[..., 0].reshape(B * N),
      boxes[..., 1].reshape(B * N),
      boxes[..., 2].reshape(B * N),
      boxes[..., 3].reshape(B * N),
      tab.reshape(B * M, 8),
      match_gt_id.astype(jnp.int32).reshape(B * N),
      match_pos_flag.astype(jnp.int32).reshape(B * N),
  )
  cls_label = cls.reshape(B, N)
  cls_label_mask = clsm.reshape(B, N)
  reg_label = jnp.stack(
      [jnp.stack([hd0.reshape(B, N), ht0.reshape(B, N)], axis=-1),
       jnp.stack([hd1.reshape(B, N), ht1.reshape(B, N)], axis=-1)], axis=2)
  rm = jnp.stack(
      [rm0.reshape(B, N) > 0.0, rm1.reshape(B, N) > 0.0], axis=2)[..., None]
  reg_label_mask = jnp.concatenate([rm, rm], axis=-1)
  return (cls_label, cls_label_mask, reg_label, reg_label_mask)
